# baseline (speedup 1.0000x reference)
import functools

import jax
import jax.numpy as jnp
from jax import lax
from jax.experimental import pallas as pl
from jax.experimental.pallas import tpu as pltpu
from jax.experimental.pallas import tpu_sc as plsc

N = 100000
XW = 512
X2W = 112
CNT_COL = 383
NPASS = 16
CH = 125
NSUB = 16
NCORE = 2
ROWS_PER_TILE = N // NSUB
NCHUNK_S = ROWS_PER_TILE // CH
N_G = 100352
CH_G = 112
ROWS_PER_W = N_G // (NSUB * NCORE)
NCHUNK_G = ROWS_PER_W // CH_G
ZROWS = 250
NBUF = 10
NGROUP = NCHUNK_S // NBUF
BR = 1000

_SRC_OF_SLICE = [(0, 16 * s) for s in range(17)] \
    + [(1, 16 * (s - 17)) for s in range(17, 24)] \
    + [(2, 16 * (s - 24)) for s in range(24, 32)]


def _build_body(ts_ref, mts_ref, tw_ref, tb_ref, x_ref):
    dt = ts_ref[...] - mts_ref[...]
    tf = jnp.cos(dt * tw_ref[...] + tb_ref[...])
    br = tf.shape[0]
    x_ref[...] = jnp.concatenate([
        tf,
        jnp.zeros((br, 11), jnp.float32),
        jnp.ones((br, 1), jnp.float32),
    ], axis=1)


def _gru_body(t_ref, wih_ref, whh_ref, bih_ref, bhh_ref, upd_ref):
    tb = t_ref[...]
    xa = tb[:, :384]
    ma = tb[:, 384:]
    cnt = tb[:, CNT_COL:CNT_COL + 1]
    hprev = ma / jnp.maximum(cnt, 1.0)
    gi = jnp.dot(xa, wih_ref[...], preferred_element_type=jnp.float32) + bih_ref[...]
    gh = jnp.dot(hprev, whh_ref[...], preferred_element_type=jnp.float32) + bhh_ref[...]
    r = jax.nn.sigmoid(gi[:, :128] + gh[:, :128])
    z = jax.nn.sigmoid(gi[:, 128:256] + gh[:, 128:256])
    n = jnp.tanh(gi[:, 256:] + r * gh[:, 256:])
    upd_ref[...] = (1.0 - z) * n + z * hprev


def _proj_body(h_ref, pw_ref, pb_ref, o_ref):
    o_ref[...] = (jnp.dot(h_ref[...], pw_ref[...],
                          preferred_element_type=jnp.float32) + pb_ref[...])


def _add_body(r_ref, p_ref, o_ref):
    o_ref[...] = r_ref[...] + p_ref[...]


_sc_mesh = plsc.VectorSubcoreMesh(core_axis_name="c", subcore_axis_name="s")
_sc_params = pltpu.CompilerParams(use_tc_tiling_on_sc=False)


@functools.partial(
    pl.kernel,
    out_type=jax.ShapeDtypeStruct((N, XW), jnp.float32),
    mesh=_sc_mesh,
    compiler_params=_sc_params,
    scratch_types=[
        pltpu.VMEM_SHARED((N, 16), jnp.float32),
        pltpu.VMEM((NCHUNK_S, CH), jnp.int32),
        pltpu.VMEM((NBUF, CH, 16), jnp.float32),
        pltpu.VMEM((ZROWS, 16), jnp.float32),
        pltpu.SemaphoreType.DMA((NBUF,)),
        pltpu.SemaphoreType.DMA((NBUF,)),
        pltpu.SemaphoreType.DMA,
    ],
)
def _scatter_kernel(mi_hbm, x2_hbm, mem_hbm, ids_hbm, table_hbm,
                    acc, ids_v, buf, zbuf, gsem, ssem, zsem):
    c = lax.axis_index("c")
    s = lax.axis_index("s")
    row0 = s * ROWS_PER_TILE
    srcs = (mi_hbm, x2_hbm, mem_hbm)
    pltpu.sync_copy(ids_hbm.at[s], ids_v)

    def zrow(i, carry):
        zbuf[i, :] = jnp.zeros((16,), jnp.float32)
        return carry
    lax.fori_loop(0, ZROWS, zrow, 0)

    def do_pass(src, scol0, dcol0):
        def xsrc(j):
            return src.at[pl.ds(row0 + j * CH, CH), pl.ds(scol0, 16)]

        for i in range(ROWS_PER_TILE // ZROWS):
            pltpu.async_copy(zbuf, acc.at[pl.ds(row0 + i * ZROWS, ZROWS), :],
                             zsem)
        for i in range(ROWS_PER_TILE // ZROWS):
            pltpu.make_async_copy(
                zbuf, acc.at[pl.ds(row0 + i * ZROWS, ZROWS), :], zsem).wait()
        plsc.subcore_barrier()

        for b in range(NBUF):
            pltpu.async_copy(xsrc(b), buf.at[b], gsem.at[b])

        def group(g, carry):
            for b in range(NBUF):
                j = g * NBUF + b
                pltpu.make_async_copy(xsrc(j), buf.at[b], gsem.at[b]).wait()
                pltpu.async_copy(buf.at[b], acc.at[ids_v.at[j]], ssem.at[b],
                                 add=True)
            for b in range(NBUF):
                j = g * NBUF + b
                pltpu.make_async_copy(buf.at[b], acc.at[ids_v.at[j]],
                                      ssem.at[b]).wait()
                jn = j + NBUF

                @pl.when(jn < NCHUNK_S)
                def _():
                    pltpu.async_copy(xsrc(jn), buf.at[b], gsem.at[b])
            return carry
        lax.fori_loop(0, NGROUP, group, 0)
        plsc.subcore_barrier()

        pltpu.sync_copy(
            acc.at[pl.ds(row0, ROWS_PER_TILE), :],
            table_hbm.at[pl.ds(row0, ROWS_PER_TILE), pl.ds(dcol0, 16)])

    for p in range(NPASS):
        sa, ca = _SRC_OF_SLICE[2 * p]
        sb, cb = _SRC_OF_SLICE[2 * p + 1]
        if sa == sb:
            do_pass(srcs[sa], ca + c * 16, (2 * p) * 16 + c * 16)
        else:
            @pl.when(c == 0)
            def _():
                do_pass(srcs[sa], ca, (2 * p) * 16)

            @pl.when(c == 1)
            def _():
                do_pass(srcs[sb], cb, (2 * p + 1) * 16)


@functools.partial(
    pl.kernel,
    out_type=jax.ShapeDtypeStruct((N_G, 128), jnp.float32),
    mesh=_sc_mesh,
    compiler_params=pltpu.CompilerParams(use_tc_tiling_on_sc=True),
    scratch_types=[
        pltpu.VMEM((NCHUNK_G, CH_G), jnp.int32),
        pltpu.VMEM((CH_G, 128), jnp.float32),
        pltpu.SemaphoreType.DMA,
    ],
)
def _gather_kernel(upd_hbm, ids_hbm, out_hbm, ids_v, rows_v, sem):
    c = lax.axis_index("c")
    s = lax.axis_index("s")
    w = s * NCORE + c
    chunk0 = w * NCHUNK_G
    pltpu.sync_copy(ids_hbm.at[w], ids_v)

    def chunk(j, carry):
        pltpu.async_copy(upd_hbm.at[ids_v.at[j]], rows_v, sem).wait()
        pltpu.sync_copy(rows_v, out_hbm.at[pl.ds((chunk0 + j) * CH_G, CH_G), :])
        return carry
    lax.fori_loop(0, NCHUNK_G, chunk, 0)


def kernel(all_ids, mem_input, ts, mem_ts, mem, h, num_dst_nodes,
           time_w, time_b, w_ih, w_hh, b_ih, b_hh, proj_w, proj_b):
    ids_i32 = all_ids.astype(jnp.int32)
    ids_s = ids_i32.reshape(NSUB, NCHUNK_S, CH)
    ids_g = jnp.pad(ids_i32, (0, N_G - N)).reshape(NSUB * NCORE, NCHUNK_G, CH_G)

    ph = pl.pallas_call(
        _proj_body,
        grid=(N // BR,),
        in_specs=[
            pl.BlockSpec((BR, 256), lambda i: (i, 0)),
            pl.BlockSpec((256, 128), lambda i: (0, 0)),
            pl.BlockSpec((1, 128), lambda i: (0, 0)),
        ],
        out_specs=pl.BlockSpec((BR, 128), lambda i: (i, 0)),
        out_shape=jax.ShapeDtypeStruct((N, 128), jnp.float32),
    )(h, proj_w.T, proj_b.reshape(1, 128))

    x2 = pl.pallas_call(
        _build_body,
        grid=(N // BR,),
        in_specs=[
            pl.BlockSpec((BR, 1), lambda i: (i, 0)),
            pl.BlockSpec((BR, 1), lambda i: (i, 0)),
            pl.BlockSpec((1, 100), lambda i: (0, 0)),
            pl.BlockSpec((1, 100), lambda i: (0, 0)),
        ],
        out_specs=pl.BlockSpec((BR, X2W), lambda i: (i, 0)),
        out_shape=jax.ShapeDtypeStruct((N, X2W), jnp.float32),
    )(ts.reshape(N, 1), mem_ts.reshape(N, 1),
      time_w.reshape(1, 100), time_b.reshape(1, 100))

    table = _scatter_kernel(mem_input, x2, mem, ids_s)

    wih_pad = jnp.zeros((384, 384), jnp.float32).at[:372, :].set(w_ih.T)
    upd = pl.pallas_call(
        _gru_body,
        grid=(N // BR,),
        in_specs=[
            pl.BlockSpec((BR, XW), lambda i: (i, 0)),
            pl.BlockSpec((384, 384), lambda i: (0, 0)),
            pl.BlockSpec((128, 384), lambda i: (0, 0)),
            pl.BlockSpec((1, 384), lambda i: (0, 0)),
            pl.BlockSpec((1, 384), lambda i: (0, 0)),
        ],
        out_specs=pl.BlockSpec((BR, 128), lambda i: (i, 0)),
        out_shape=jax.ShapeDtypeStruct((N, 128), jnp.float32),
    )(table, wih_pad, w_hh.T, b_ih.reshape(1, 384), b_hh.reshape(1, 384))

    restored = _gather_kernel(upd, ids_g)

    h_out = pl.pallas_call(
        _add_body,
        grid=(N // BR,),
        in_specs=[
            pl.BlockSpec((BR, 128), lambda i: (i, 0)),
            pl.BlockSpec((BR, 128), lambda i: (i, 0)),
        ],
        out_specs=pl.BlockSpec((BR, 128), lambda i: (i, 0)),
        out_shape=jax.ShapeDtypeStruct((N, 128), jnp.float32),
    )(restored, ph)

    nd = 50000
    last_updated_nid = all_ids[:nd] + (num_dst_nodes - nd)
    return last_updated_nid, restored[:nd], ts[:nd], h_out


# speedup vs baseline: 1.1174x; 1.1174x over previous
import functools

import jax
import jax.numpy as jnp
from jax import lax
from jax.experimental import pallas as pl
from jax.experimental.pallas import tpu as pltpu
from jax.experimental.pallas import tpu_sc as plsc

N = 100000
XW = 512
X2W = 128
CNT_COL = 383
NPASS = 16
CH = 125
NSUB = 16
NCORE = 2
ROWS_PER_TILE = N // NSUB
NCHUNK_S = ROWS_PER_TILE // CH
N_G = 100352
CH_G = 112
ROWS_PER_W = N_G // (NSUB * NCORE)
NCHUNK_G = ROWS_PER_W // CH_G
ZROWS = 250
NBUF = 10
NGROUP = NCHUNK_S // NBUF
BR = 1000

_SRC_OF_SLICE = [(0, 16 * s) for s in range(17)] \
    + [(1, 16 * (s - 17)) for s in range(17, 24)] \
    + [(2, 16 * (s - 24)) for s in range(24, 32)]


def _build_body(ts_ref, mts_ref, tw_ref, tb_ref, x_ref):
    dt = ts_ref[...] - mts_ref[...]
    tf = jnp.cos(dt * tw_ref[...] + tb_ref[...])
    br = tf.shape[0]
    x_ref[...] = jnp.concatenate([
        tf,
        jnp.zeros((br, 11), jnp.float32),
        jnp.ones((br, 1), jnp.float32),
        jnp.zeros((br, 16), jnp.float32),
    ], axis=1)


def _gru_body(t_ref, wih_ref, whh_ref, bih_ref, bhh_ref, upd_ref):
    t = t_ref[...]
    xa = jnp.concatenate([t[0], t[1], t[2]], axis=1)
    ma = t[3]
    cnt = t[2][:, 127:128]
    hprev = ma / jnp.maximum(cnt, 1.0)
    gi = jnp.dot(xa, wih_ref[...], preferred_element_type=jnp.float32) + bih_ref[...]
    gh = jnp.dot(hprev, whh_ref[...], preferred_element_type=jnp.float32) + bhh_ref[...]
    r = jax.nn.sigmoid(gi[:, :128] + gh[:, :128])
    z = jax.nn.sigmoid(gi[:, 128:256] + gh[:, 128:256])
    n = jnp.tanh(gi[:, 256:] + r * gh[:, 256:])
    upd_ref[...] = (1.0 - z) * n + z * hprev


def _proj_body(h_ref, pw_ref, pb_ref, o_ref):
    o_ref[...] = (jnp.dot(h_ref[...], pw_ref[...],
                          preferred_element_type=jnp.float32) + pb_ref[...])


def _add_body(r_ref, p_ref, o_ref):
    o_ref[...] = r_ref[...] + p_ref[...]


_sc_mesh = plsc.VectorSubcoreMesh(core_axis_name="c", subcore_axis_name="s")
_sc_params = pltpu.CompilerParams(use_tc_tiling_on_sc=False)


@functools.partial(
    pl.kernel,
    out_type=jax.ShapeDtypeStruct((4, N, 128), jnp.float32),
    mesh=_sc_mesh,
    compiler_params=_sc_params,
    scratch_types=[
        pltpu.VMEM_SHARED((N, 16), jnp.float32),
        pltpu.VMEM((NCHUNK_S, CH), jnp.int32),
        pltpu.VMEM((NBUF, CH, 16), jnp.float32),
        pltpu.VMEM((ZROWS, 16), jnp.float32),
        pltpu.SemaphoreType.DMA((NBUF,)),
        pltpu.SemaphoreType.DMA((NBUF,)),
        pltpu.SemaphoreType.DMA,
    ],
)
def _scatter_kernel(mi_hbm, x2_hbm, mem_hbm, ids_hbm, table_hbm,
                    acc, ids_v, buf, zbuf, gsem, ssem, zsem):
    c = lax.axis_index("c")
    s = lax.axis_index("s")
    row0 = s * ROWS_PER_TILE
    srcs = (mi_hbm, x2_hbm, mem_hbm)
    pltpu.sync_copy(ids_hbm.at[s], ids_v)

    def zrow(i, carry):
        zbuf[i, :] = jnp.zeros((16,), jnp.float32)
        return carry
    lax.fori_loop(0, ZROWS, zrow, 0)

    def do_pass(src, scol0, dcol0):
        def xsrc(j):
            return src.at[pl.ds(row0 + j * CH, CH), pl.ds(scol0, 16)]

        for i in range(ROWS_PER_TILE // ZROWS):
            pltpu.async_copy(zbuf, acc.at[pl.ds(row0 + i * ZROWS, ZROWS), :],
                             zsem)
        for i in range(ROWS_PER_TILE // ZROWS):
            pltpu.make_async_copy(
                zbuf, acc.at[pl.ds(row0 + i * ZROWS, ZROWS), :], zsem).wait()
        plsc.subcore_barrier()

        for b in range(NBUF):
            pltpu.async_copy(xsrc(b), buf.at[b], gsem.at[b])

        def group(g, carry):
            for b in range(NBUF):
                j = g * NBUF + b
                pltpu.make_async_copy(xsrc(j), buf.at[b], gsem.at[b]).wait()
                pltpu.async_copy(buf.at[b], acc.at[ids_v.at[j]], ssem.at[b],
                                 add=True)
            for b in range(NBUF):
                j = g * NBUF + b
                pltpu.make_async_copy(buf.at[b], acc.at[ids_v.at[j]],
                                      ssem.at[b]).wait()
                jn = j + NBUF

                @pl.when(jn < NCHUNK_S)
                def _():
                    pltpu.async_copy(xsrc(jn), buf.at[b], gsem.at[b])
            return carry
        lax.fori_loop(0, NGROUP, group, 0)
        plsc.subcore_barrier()

        pltpu.sync_copy(
            acc.at[pl.ds(row0, ROWS_PER_TILE), :],
            table_hbm.at[slab, pl.ds(row0, ROWS_PER_TILE), pl.ds(dcol0, 16)])

    for p in range(NPASS):
        slab = p // 4
        sa, ca = _SRC_OF_SLICE[2 * p]
        sb, cb = _SRC_OF_SLICE[2 * p + 1]
        if sa == sb:
            do_pass(srcs[sa], ca + c * 16, ((2 * p) % 8 + c) * 16)
        else:
            @pl.when(c == 0)
            def _():
                do_pass(srcs[sa], ca, ((2 * p) % 8) * 16)

            @pl.when(c == 1)
            def _():
                do_pass(srcs[sb], cb, ((2 * p) % 8 + 1) * 16)


@functools.partial(
    pl.kernel,
    out_type=jax.ShapeDtypeStruct((N_G, 128), jnp.float32),
    mesh=_sc_mesh,
    compiler_params=pltpu.CompilerParams(use_tc_tiling_on_sc=True),
    scratch_types=[
        pltpu.VMEM((NCHUNK_G, CH_G), jnp.int32),
        pltpu.VMEM((CH_G, 128), jnp.float32),
        pltpu.SemaphoreType.DMA,
    ],
)
def _gather_kernel(upd_hbm, ids_hbm, out_hbm, ids_v, rows_v, sem):
    c = lax.axis_index("c")
    s = lax.axis_index("s")
    w = s * NCORE + c
    chunk0 = w * NCHUNK_G
    pltpu.sync_copy(ids_hbm.at[w], ids_v)

    def chunk(j, carry):
        pltpu.async_copy(upd_hbm.at[ids_v.at[j]], rows_v, sem).wait()
        pltpu.sync_copy(rows_v, out_hbm.at[pl.ds((chunk0 + j) * CH_G, CH_G), :])
        return carry
    lax.fori_loop(0, NCHUNK_G, chunk, 0)


def kernel(all_ids, mem_input, ts, mem_ts, mem, h, num_dst_nodes,
           time_w, time_b, w_ih, w_hh, b_ih, b_hh, proj_w, proj_b):
    ids_i32 = all_ids.astype(jnp.int32)
    ids_s = ids_i32.reshape(NSUB, NCHUNK_S, CH)
    ids_g = jnp.pad(ids_i32, (0, N_G - N)).reshape(NSUB * NCORE, NCHUNK_G, CH_G)

    ph = pl.pallas_call(
        _proj_body,
        grid=(N // BR,),
        in_specs=[
            pl.BlockSpec((BR, 256), lambda i: (i, 0)),
            pl.BlockSpec((256, 128), lambda i: (0, 0)),
            pl.BlockSpec((1, 128), lambda i: (0, 0)),
        ],
        out_specs=pl.BlockSpec((BR, 128), lambda i: (i, 0)),
        out_shape=jax.ShapeDtypeStruct((N, 128), jnp.float32),
    )(h, proj_w.T, proj_b.reshape(1, 128))

    x2 = pl.pallas_call(
        _build_body,
        grid=(N // BR,),
        in_specs=[
            pl.BlockSpec((BR, 1), lambda i: (i, 0)),
            pl.BlockSpec((BR, 1), lambda i: (i, 0)),
            pl.BlockSpec((1, 100), lambda i: (0, 0)),
            pl.BlockSpec((1, 100), lambda i: (0, 0)),
        ],
        out_specs=pl.BlockSpec((BR, X2W), lambda i: (i, 0)),
        out_shape=jax.ShapeDtypeStruct((N, X2W), jnp.float32),
    )(ts.reshape(N, 1), mem_ts.reshape(N, 1),
      time_w.reshape(1, 100), time_b.reshape(1, 100))

    table = _scatter_kernel(mem_input, x2, mem, ids_s)

    wih_pad = jnp.zeros((384, 384), jnp.float32).at[:372, :].set(w_ih.T)
    upd = pl.pallas_call(
        _gru_body,
        grid=(N // BR,),
        in_specs=[
            pl.BlockSpec((4, BR, 128), lambda i: (0, i, 0)),
            pl.BlockSpec((384, 384), lambda i: (0, 0)),
            pl.BlockSpec((128, 384), lambda i: (0, 0)),
            pl.BlockSpec((1, 384), lambda i: (0, 0)),
            pl.BlockSpec((1, 384), lambda i: (0, 0)),
        ],
        out_specs=pl.BlockSpec((BR, 128), lambda i: (i, 0)),
        out_shape=jax.ShapeDtypeStruct((N, 128), jnp.float32),
    )(table, wih_pad, w_hh.T, b_ih.reshape(1, 384), b_hh.reshape(1, 384))

    restored = _gather_kernel(upd, ids_g)

    h_out = pl.pallas_call(
        _add_body,
        grid=(N // BR,),
        in_specs=[
            pl.BlockSpec((BR, 128), lambda i: (i, 0)),
            pl.BlockSpec((BR, 128), lambda i: (i, 0)),
        ],
        out_specs=pl.BlockSpec((BR, 128), lambda i: (i, 0)),
        out_shape=jax.ShapeDtypeStruct((N, 128), jnp.float32),
    )(restored, ph)

    nd = 50000
    last_updated_nid = all_ids[:nd] + (num_dst_nodes - nd)
    return last_updated_nid, restored[:nd], ts[:nd], h_out


# speedup vs baseline: 1.1183x; 1.0007x over previous
import functools

import jax
import jax.numpy as jnp
from jax import lax
from jax.experimental import pallas as pl
from jax.experimental.pallas import tpu as pltpu
from jax.experimental.pallas import tpu_sc as plsc

N = 100000
XW = 512
X2W = 128
CNT_COL = 383
NPASS = 16
CH = 125
NSUB = 16
NCORE = 2
ROWS_PER_TILE = N // NSUB
NCHUNK_S = ROWS_PER_TILE // CH
N_G = 100352
CH_G = 112
ROWS_PER_W = N_G // (NSUB * NCORE)
NCHUNK_G = ROWS_PER_W // CH_G
ZROWS = 250
NBUF = 10
NGROUP = NCHUNK_S // NBUF
BR = 1000

_SRC_OF_SLICE = [(0, 16 * s) for s in range(17)] \
    + [(1, 16 * (s - 17)) for s in range(17, 24)] \
    + [(2, 16 * (s - 24)) for s in range(24, 32)]


def _build_body(ts_ref, mts_ref, tw_ref, tb_ref, x_ref):
    dt = ts_ref[...] - mts_ref[...]
    tf = jnp.cos(dt * tw_ref[...] + tb_ref[...])
    br = tf.shape[0]
    x_ref[...] = jnp.concatenate([
        tf,
        jnp.zeros((br, 11), jnp.float32),
        jnp.ones((br, 1), jnp.float32),
        jnp.zeros((br, 16), jnp.float32),
    ], axis=1)


def _gru_body(t_ref, wih_ref, whh_ref, bih_ref, bhh_ref, upd_ref):
    t = t_ref[...]
    xa = jnp.concatenate([t[0], t[1], t[2]], axis=1)
    ma = t[3]
    cnt = t[2][:, 127:128]
    hprev = ma / jnp.maximum(cnt, 1.0)
    gi = jnp.dot(xa.astype(jnp.bfloat16), wih_ref[...],
                 preferred_element_type=jnp.float32) + bih_ref[...]
    gh = jnp.dot(hprev.astype(jnp.bfloat16), whh_ref[...],
                 preferred_element_type=jnp.float32) + bhh_ref[...]
    r = jax.nn.sigmoid(gi[:, :128] + gh[:, :128])
    z = jax.nn.sigmoid(gi[:, 128:256] + gh[:, 128:256])
    n = jnp.tanh(gi[:, 256:] + r * gh[:, 256:])
    upd_ref[...] = (1.0 - z) * n + z * hprev


def _proj_body(h_ref, pw_ref, pb_ref, o_ref):
    o_ref[...] = (jnp.dot(h_ref[...].astype(jnp.bfloat16), pw_ref[...],
                          preferred_element_type=jnp.float32) + pb_ref[...])


def _add_body(r_ref, p_ref, o_ref):
    o_ref[...] = r_ref[...] + p_ref[...]


_sc_mesh = plsc.VectorSubcoreMesh(core_axis_name="c", subcore_axis_name="s")
_sc_params = pltpu.CompilerParams(use_tc_tiling_on_sc=False)


@functools.partial(
    pl.kernel,
    out_type=jax.ShapeDtypeStruct((4, N, 128), jnp.float32),
    mesh=_sc_mesh,
    compiler_params=_sc_params,
    scratch_types=[
        pltpu.VMEM_SHARED((N, 16), jnp.float32),
        pltpu.VMEM((NCHUNK_S, CH), jnp.int32),
        pltpu.VMEM((NBUF, CH, 16), jnp.float32),
        pltpu.VMEM((ZROWS, 16), jnp.float32),
        pltpu.SemaphoreType.DMA((NBUF,)),
        pltpu.SemaphoreType.DMA((NBUF,)),
        pltpu.SemaphoreType.DMA,
    ],
)
def _scatter_kernel(mi_hbm, x2_hbm, mem_hbm, ids_hbm, table_hbm,
                    acc, ids_v, buf, zbuf, gsem, ssem, zsem):
    c = lax.axis_index("c")
    s = lax.axis_index("s")
    row0 = s * ROWS_PER_TILE
    srcs = (mi_hbm, x2_hbm, mem_hbm)
    pltpu.sync_copy(ids_hbm.at[s], ids_v)

    def zrow(i, carry):
        zbuf[i, :] = jnp.zeros((16,), jnp.float32)
        return carry
    lax.fori_loop(0, ZROWS, zrow, 0)

    def do_pass(src, scol0, dcol0):
        def xsrc(j):
            return src.at[pl.ds(row0 + j * CH, CH), pl.ds(scol0, 16)]

        for i in range(ROWS_PER_TILE // ZROWS):
            pltpu.async_copy(zbuf, acc.at[pl.ds(row0 + i * ZROWS, ZROWS), :],
                             zsem)
        for i in range(ROWS_PER_TILE // ZROWS):
            pltpu.make_async_copy(
                zbuf, acc.at[pl.ds(row0 + i * ZROWS, ZROWS), :], zsem).wait()
        plsc.subcore_barrier()

        for b in range(NBUF):
            pltpu.async_copy(xsrc(b), buf.at[b], gsem.at[b])

        def group(g, carry):
            for b in range(NBUF):
                j = g * NBUF + b
                pltpu.make_async_copy(xsrc(j), buf.at[b], gsem.at[b]).wait()
                pltpu.async_copy(buf.at[b], acc.at[ids_v.at[j]], ssem.at[b],
                                 add=True)
            for b in range(NBUF):
                j = g * NBUF + b
                pltpu.make_async_copy(buf.at[b], acc.at[ids_v.at[j]],
                                      ssem.at[b]).wait()
                jn = j + NBUF

                @pl.when(jn < NCHUNK_S)
                def _():
                    pltpu.async_copy(xsrc(jn), buf.at[b], gsem.at[b])
            return carry
        lax.fori_loop(0, NGROUP, group, 0)
        plsc.subcore_barrier()

        pltpu.sync_copy(
            acc.at[pl.ds(row0, ROWS_PER_TILE), :],
            table_hbm.at[slab, pl.ds(row0, ROWS_PER_TILE), pl.ds(dcol0, 16)])

    for p in range(NPASS):
        slab = p // 4
        sa, ca = _SRC_OF_SLICE[2 * p]
        sb, cb = _SRC_OF_SLICE[2 * p + 1]
        if sa == sb:
            do_pass(srcs[sa], ca + c * 16, ((2 * p) % 8 + c) * 16)
        else:
            @pl.when(c == 0)
            def _():
                do_pass(srcs[sa], ca, ((2 * p) % 8) * 16)

            @pl.when(c == 1)
            def _():
                do_pass(srcs[sb], cb, ((2 * p) % 8 + 1) * 16)


@functools.partial(
    pl.kernel,
    out_type=jax.ShapeDtypeStruct((N_G, 128), jnp.float32),
    mesh=_sc_mesh,
    compiler_params=pltpu.CompilerParams(use_tc_tiling_on_sc=True),
    scratch_types=[
        pltpu.VMEM((NCHUNK_G, CH_G), jnp.int32),
        pltpu.VMEM((CH_G, 128), jnp.float32),
        pltpu.SemaphoreType.DMA,
    ],
)
def _gather_kernel(upd_hbm, ids_hbm, out_hbm, ids_v, rows_v, sem):
    c = lax.axis_index("c")
    s = lax.axis_index("s")
    w = s * NCORE + c
    chunk0 = w * NCHUNK_G
    pltpu.sync_copy(ids_hbm.at[w], ids_v)

    def chunk(j, carry):
        pltpu.async_copy(upd_hbm.at[ids_v.at[j]], rows_v, sem).wait()
        pltpu.sync_copy(rows_v, out_hbm.at[pl.ds((chunk0 + j) * CH_G, CH_G), :])
        return carry
    lax.fori_loop(0, NCHUNK_G, chunk, 0)


def kernel(all_ids, mem_input, ts, mem_ts, mem, h, num_dst_nodes,
           time_w, time_b, w_ih, w_hh, b_ih, b_hh, proj_w, proj_b):
    ids_i32 = all_ids.astype(jnp.int32)
    ids_s = ids_i32.reshape(NSUB, NCHUNK_S, CH)
    ids_g = jnp.pad(ids_i32, (0, N_G - N)).reshape(NSUB * NCORE, NCHUNK_G, CH_G)

    ph = pl.pallas_call(
        _proj_body,
        grid=(N // BR,),
        in_specs=[
            pl.BlockSpec((BR, 256), lambda i: (i, 0)),
            pl.BlockSpec((256, 128), lambda i: (0, 0)),
            pl.BlockSpec((1, 128), lambda i: (0, 0)),
        ],
        out_specs=pl.BlockSpec((BR, 128), lambda i: (i, 0)),
        out_shape=jax.ShapeDtypeStruct((N, 128), jnp.float32),
    )(h, proj_w.T.astype(jnp.bfloat16), proj_b.reshape(1, 128))

    x2 = pl.pallas_call(
        _build_body,
        grid=(N // BR,),
        in_specs=[
            pl.BlockSpec((BR, 1), lambda i: (i, 0)),
            pl.BlockSpec((BR, 1), lambda i: (i, 0)),
            pl.BlockSpec((1, 100), lambda i: (0, 0)),
            pl.BlockSpec((1, 100), lambda i: (0, 0)),
        ],
        out_specs=pl.BlockSpec((BR, X2W), lambda i: (i, 0)),
        out_shape=jax.ShapeDtypeStruct((N, X2W), jnp.float32),
    )(ts.reshape(N, 1), mem_ts.reshape(N, 1),
      time_w.reshape(1, 100), time_b.reshape(1, 100))

    table = _scatter_kernel(mem_input, x2, mem, ids_s)

    wih_pad = jnp.zeros((384, 384), jnp.float32).at[:372, :].set(w_ih.T)
    upd = pl.pallas_call(
        _gru_body,
        grid=(N // BR,),
        in_specs=[
            pl.BlockSpec((4, BR, 128), lambda i: (0, i, 0)),
            pl.BlockSpec((384, 384), lambda i: (0, 0)),
            pl.BlockSpec((128, 384), lambda i: (0, 0)),
            pl.BlockSpec((1, 384), lambda i: (0, 0)),
            pl.BlockSpec((1, 384), lambda i: (0, 0)),
        ],
        out_specs=pl.BlockSpec((BR, 128), lambda i: (i, 0)),
        out_shape=jax.ShapeDtypeStruct((N, 128), jnp.float32),
    )(table, wih_pad.astype(jnp.bfloat16), w_hh.T.astype(jnp.bfloat16),
      b_ih.reshape(1, 384), b_hh.reshape(1, 384))

    restored = _gather_kernel(upd, ids_g)

    h_out = pl.pallas_call(
        _add_body,
        grid=(N // BR,),
        in_specs=[
            pl.BlockSpec((BR, 128), lambda i: (i, 0)),
            pl.BlockSpec((BR, 128), lambda i: (i, 0)),
        ],
        out_specs=pl.BlockSpec((BR, 128), lambda i: (i, 0)),
        out_shape=jax.ShapeDtypeStruct((N, 128), jnp.float32),
    )(restored, ph)

    nd = 50000
    last_updated_nid = all_ids[:nd] + (num_dst_nodes - nd)
    return last_updated_nid, restored[:nd], ts[:nd], h_out


# speedup vs baseline: 1.2510x; 1.1187x over previous
import functools

import jax
import jax.numpy as jnp
from jax import lax
from jax.experimental import pallas as pl
from jax.experimental.pallas import tpu as pltpu
from jax.experimental.pallas import tpu_sc as plsc

N = 100000
XW = 512
X2W = 128
CNT_COL = 383
NPASS = 16
CH = 125
NSUB = 16
NCORE = 2
ROWS_PER_TILE = N // NSUB
NCHUNK_S = ROWS_PER_TILE // CH
N_G = 100352
CH_G = 112
ROWS_PER_W = N_G // (NSUB * NCORE)
NCHUNK_G = ROWS_PER_W // CH_G
ZROWS = 250
NBUF = 10
NGROUP = NCHUNK_S // NBUF
BR = 1000

_SRC_OF_SLICE = [(s // 8, 16 * (s % 8)) for s in range(32)]


def _build_body(mi_ref, ts_ref, mts_ref, tw_ref, tb_ref, mem_ref,
                a_ref, b_ref, x_ref, m_ref):
    mi = mi_ref[...]
    a_ref[...] = mi[:, :128]
    b_ref[...] = mi[:, 128:256]
    dt = ts_ref[...] - mts_ref[...]
    tf = jnp.cos(dt * tw_ref[...] + tb_ref[...])
    br = tf.shape[0]
    x_ref[...] = jnp.concatenate([
        mi[:, 256:272],
        tf,
        jnp.zeros((br, 11), jnp.float32),
        jnp.ones((br, 1), jnp.float32),
    ], axis=1)
    m_ref[...] = mem_ref[...]


def _gru_body(t_ref, wih_ref, whh_ref, bih_ref, bhh_ref, upd_ref):
    t = t_ref[...]
    xa = jnp.concatenate([t[0], t[1], t[2]], axis=1)
    ma = t[3]
    cnt = t[2][:, 127:128]
    hprev = ma / jnp.maximum(cnt, 1.0)
    gi = jnp.dot(xa.astype(jnp.bfloat16), wih_ref[...],
                 preferred_element_type=jnp.float32) + bih_ref[...]
    gh = jnp.dot(hprev.astype(jnp.bfloat16), whh_ref[...],
                 preferred_element_type=jnp.float32) + bhh_ref[...]
    r = jax.nn.sigmoid(gi[:, :128] + gh[:, :128])
    z = jax.nn.sigmoid(gi[:, 128:256] + gh[:, 128:256])
    n = jnp.tanh(gi[:, 256:] + r * gh[:, 256:])
    upd_ref[...] = (1.0 - z) * n + z * hprev


def _proj_body(h_ref, pw_ref, pb_ref, o_ref):
    o_ref[...] = (jnp.dot(h_ref[...].astype(jnp.bfloat16), pw_ref[...],
                          preferred_element_type=jnp.float32) + pb_ref[...])


def _add_body(r_ref, p_ref, o_ref):
    o_ref[...] = r_ref[...] + p_ref[...]


_sc_mesh = plsc.VectorSubcoreMesh(core_axis_name="c", subcore_axis_name="s")
_sc_params = pltpu.CompilerParams(use_tc_tiling_on_sc=False)


@functools.partial(
    pl.kernel,
    out_type=jax.ShapeDtypeStruct((4, N, 128), jnp.float32),
    mesh=_sc_mesh,
    compiler_params=_sc_params,
    scratch_types=[
        pltpu.VMEM_SHARED((N, 16), jnp.float32),
        pltpu.VMEM((NCHUNK_S, CH), jnp.int32),
        pltpu.VMEM((NBUF, CH, 16), jnp.float32),
        pltpu.VMEM((ZROWS, 16), jnp.float32),
        pltpu.SemaphoreType.DMA((NBUF,)),
        pltpu.SemaphoreType.DMA((NBUF,)),
        pltpu.SemaphoreType.DMA,
    ],
)
def _scatter_kernel(a_hbm, b_hbm, x2_hbm, mem_hbm, ids_hbm, table_hbm,
                    acc, ids_v, buf, zbuf, gsem, ssem, zsem):
    c = lax.axis_index("c")
    s = lax.axis_index("s")
    row0 = s * ROWS_PER_TILE
    srcs = (a_hbm, b_hbm, x2_hbm, mem_hbm)
    pltpu.sync_copy(ids_hbm.at[s], ids_v)

    def zrow(i, carry):
        zbuf[i, :] = jnp.zeros((16,), jnp.float32)
        return carry
    lax.fori_loop(0, ZROWS, zrow, 0)

    def do_pass(src, scol0, dcol0):
        def xsrc(j):
            return src.at[pl.ds(row0 + j * CH, CH), pl.ds(scol0, 16)]

        for i in range(ROWS_PER_TILE // ZROWS):
            pltpu.async_copy(zbuf, acc.at[pl.ds(row0 + i * ZROWS, ZROWS), :],
                             zsem)
        for i in range(ROWS_PER_TILE // ZROWS):
            pltpu.make_async_copy(
                zbuf, acc.at[pl.ds(row0 + i * ZROWS, ZROWS), :], zsem).wait()
        plsc.subcore_barrier()

        for b in range(NBUF):
            pltpu.async_copy(xsrc(b), buf.at[b], gsem.at[b])

        def group(g, carry):
            for b in range(NBUF):
                j = g * NBUF + b
                pltpu.make_async_copy(xsrc(j), buf.at[b], gsem.at[b]).wait()
                pltpu.async_copy(buf.at[b], acc.at[ids_v.at[j]], ssem.at[b],
                                 add=True)
            for b in range(NBUF):
                j = g * NBUF + b
                pltpu.make_async_copy(buf.at[b], acc.at[ids_v.at[j]],
                                      ssem.at[b]).wait()
                jn = j + NBUF

                @pl.when(jn < NCHUNK_S)
                def _():
                    pltpu.async_copy(xsrc(jn), buf.at[b], gsem.at[b])
            return carry
        lax.fori_loop(0, NGROUP, group, 0)
        plsc.subcore_barrier()

        pltpu.sync_copy(
            acc.at[pl.ds(row0, ROWS_PER_TILE), :],
            table_hbm.at[slab, pl.ds(row0, ROWS_PER_TILE), pl.ds(dcol0, 16)])

    for p in range(NPASS):
        slab = p // 4
        sa, ca = _SRC_OF_SLICE[2 * p]
        do_pass(srcs[sa], ca + c * 16, ((2 * p) % 8 + c) * 16)


@functools.partial(
    pl.kernel,
    out_type=jax.ShapeDtypeStruct((N_G, 128), jnp.float32),
    mesh=_sc_mesh,
    compiler_params=pltpu.CompilerParams(use_tc_tiling_on_sc=True),
    scratch_types=[
        pltpu.VMEM((NCHUNK_G, CH_G), jnp.int32),
        pltpu.VMEM((CH_G, 128), jnp.float32),
        pltpu.SemaphoreType.DMA,
    ],
)
def _gather_kernel(upd_hbm, ids_hbm, out_hbm, ids_v, rows_v, sem):
    c = lax.axis_index("c")
    s = lax.axis_index("s")
    w = s * NCORE + c
    chunk0 = w * NCHUNK_G
    pltpu.sync_copy(ids_hbm.at[w], ids_v)

    def chunk(j, carry):
        pltpu.async_copy(upd_hbm.at[ids_v.at[j]], rows_v, sem).wait()
        pltpu.sync_copy(rows_v, out_hbm.at[pl.ds((chunk0 + j) * CH_G, CH_G), :])
        return carry
    lax.fori_loop(0, NCHUNK_G, chunk, 0)


def kernel(all_ids, mem_input, ts, mem_ts, mem, h, num_dst_nodes,
           time_w, time_b, w_ih, w_hh, b_ih, b_hh, proj_w, proj_b):
    ids_i32 = all_ids.astype(jnp.int32)
    ids_s = ids_i32.reshape(NSUB, NCHUNK_S, CH)
    ids_g = jnp.pad(ids_i32, (0, N_G - N)).reshape(NSUB * NCORE, NCHUNK_G, CH_G)

    ph = pl.pallas_call(
        _proj_body,
        grid=(N // BR,),
        in_specs=[
            pl.BlockSpec((BR, 256), lambda i: (i, 0)),
            pl.BlockSpec((256, 128), lambda i: (0, 0)),
            pl.BlockSpec((1, 128), lambda i: (0, 0)),
        ],
        out_specs=pl.BlockSpec((BR, 128), lambda i: (i, 0)),
        out_shape=jax.ShapeDtypeStruct((N, 128), jnp.float32),
    )(h, proj_w.T.astype(jnp.bfloat16), proj_b.reshape(1, 128))

    slab_spec = pl.BlockSpec((BR, 128), lambda i: (i, 0))
    slab_shape = jax.ShapeDtypeStruct((N, 128), jnp.float32)
    mi_a, mi_b, x2, mem_c = pl.pallas_call(
        _build_body,
        grid=(N // BR,),
        in_specs=[
            pl.BlockSpec((BR, 272), lambda i: (i, 0)),
            pl.BlockSpec((BR, 1), lambda i: (i, 0)),
            pl.BlockSpec((BR, 1), lambda i: (i, 0)),
            pl.BlockSpec((1, 100), lambda i: (0, 0)),
            pl.BlockSpec((1, 100), lambda i: (0, 0)),
            pl.BlockSpec((BR, 128), lambda i: (i, 0)),
        ],
        out_specs=[slab_spec, slab_spec, slab_spec, slab_spec],
        out_shape=[slab_shape, slab_shape, slab_shape, slab_shape],
    )(mem_input, ts.reshape(N, 1), mem_ts.reshape(N, 1),
      time_w.reshape(1, 100), time_b.reshape(1, 100), mem)

    table = _scatter_kernel(mi_a, mi_b, x2, mem_c, ids_s)

    wih_pad = jnp.zeros((384, 384), jnp.float32).at[:372, :].set(w_ih.T)
    upd = pl.pallas_call(
        _gru_body,
        grid=(N // BR,),
        in_specs=[
            pl.BlockSpec((4, BR, 128), lambda i: (0, i, 0)),
            pl.BlockSpec((384, 384), lambda i: (0, 0)),
            pl.BlockSpec((128, 384), lambda i: (0, 0)),
            pl.BlockSpec((1, 384), lambda i: (0, 0)),
            pl.BlockSpec((1, 384), lambda i: (0, 0)),
        ],
        out_specs=pl.BlockSpec((BR, 128), lambda i: (i, 0)),
        out_shape=jax.ShapeDtypeStruct((N, 128), jnp.float32),
    )(table, wih_pad.astype(jnp.bfloat16), w_hh.T.astype(jnp.bfloat16),
      b_ih.reshape(1, 384), b_hh.reshape(1, 384))

    restored = _gather_kernel(upd, ids_g)

    h_out = pl.pallas_call(
        _add_body,
        grid=(N // BR,),
        in_specs=[
            pl.BlockSpec((BR, 128), lambda i: (i, 0)),
            pl.BlockSpec((BR, 128), lambda i: (i, 0)),
        ],
        out_specs=pl.BlockSpec((BR, 128), lambda i: (i, 0)),
        out_shape=jax.ShapeDtypeStruct((N, 128), jnp.float32),
    )(restored, ph)

    nd = 50000
    last_updated_nid = all_ids[:nd] + (num_dst_nodes - nd)
    return last_updated_nid, restored[:nd], ts[:nd], h_out


# speedup vs baseline: 1.2561x; 1.0041x over previous
import functools

import jax
import jax.numpy as jnp
from jax import lax
from jax.experimental import pallas as pl
from jax.experimental.pallas import tpu as pltpu
from jax.experimental.pallas import tpu_sc as plsc

N = 100000
XW = 512
X2W = 128
CNT_COL = 383
NPASS = 16
CH = 125
NSUB = 16
NCORE = 2
ROWS_PER_TILE = N // NSUB
NCHUNK_S = ROWS_PER_TILE // CH
N_G = 100352
CH_G = 112
ROWS_PER_W = N_G // (NSUB * NCORE)
NCHUNK_G = ROWS_PER_W // CH_G
ZROWS = 250
NBUF = 10
NGROUP = NCHUNK_S // NBUF
BR = 1000

_SRC_OF_SLICE = [(s // 8, 16 * (s % 8)) for s in range(32)]


def _build_body(mi_ref, ts_ref, mts_ref, tw_ref, tb_ref, mem_ref,
                a_ref, b_ref, x_ref, m_ref):
    mi = mi_ref[...]
    a_ref[...] = mi[:, :128]
    b_ref[...] = mi[:, 128:256]
    dt = ts_ref[...] - mts_ref[...]
    tf = jnp.cos(dt * tw_ref[...] + tb_ref[...])
    br = tf.shape[0]
    x_ref[...] = jnp.concatenate([
        mi[:, 256:272],
        tf,
        jnp.zeros((br, 11), jnp.float32),
        jnp.ones((br, 1), jnp.float32),
    ], axis=1)
    m_ref[...] = mem_ref[...]


def _gru_body(t_ref, wih_ref, whh_ref, bih_ref, bhh_ref, upd_ref):
    t = t_ref[...]
    xa = jnp.concatenate([t[0], t[1], t[2]], axis=1)
    ma = t[3]
    cnt = t[2][:, 127:128]
    hprev = ma / jnp.maximum(cnt, 1.0)
    gi = jnp.dot(xa.astype(jnp.bfloat16), wih_ref[...],
                 preferred_element_type=jnp.float32) + bih_ref[...]
    gh = jnp.dot(hprev.astype(jnp.bfloat16), whh_ref[...],
                 preferred_element_type=jnp.float32) + bhh_ref[...]
    r = jax.nn.sigmoid(gi[:, :128] + gh[:, :128])
    z = jax.nn.sigmoid(gi[:, 128:256] + gh[:, 128:256])
    n = jnp.tanh(gi[:, 256:] + r * gh[:, 256:])
    upd_ref[...] = (1.0 - z) * n + z * hprev


def _proj_body(h_ref, pw_ref, pb_ref, o_ref):
    o_ref[...] = (jnp.dot(h_ref[...].astype(jnp.bfloat16), pw_ref[...],
                          preferred_element_type=jnp.float32) + pb_ref[...])


def _add_body(r_ref, p_ref, o_ref):
    o_ref[...] = r_ref[...] + p_ref[...]


_sc_mesh = plsc.VectorSubcoreMesh(core_axis_name="c", subcore_axis_name="s")
_sc_params = pltpu.CompilerParams(use_tc_tiling_on_sc=False)


@functools.partial(
    pl.kernel,
    out_type=jax.ShapeDtypeStruct((4, N, 128), jnp.float32),
    mesh=_sc_mesh,
    compiler_params=_sc_params,
    scratch_types=[
        pltpu.VMEM_SHARED((N, 16), jnp.float32),
        pltpu.VMEM((NCHUNK_S, CH), jnp.int32),
        pltpu.VMEM((NBUF, CH, 16), jnp.float32),
        pltpu.VMEM((ZROWS, 16), jnp.float32),
        pltpu.SemaphoreType.DMA((NBUF,)),
        pltpu.SemaphoreType.DMA((NBUF,)),
        pltpu.SemaphoreType.DMA,
        pltpu.SemaphoreType.DMA,
    ],
)
def _scatter_kernel(a_hbm, b_hbm, x2_hbm, mem_hbm, ids_hbm, table_hbm,
                    acc, ids_v, buf, zbuf, gsem, ssem, zsem, fsem):
    c = lax.axis_index("c")
    s = lax.axis_index("s")
    row0 = s * ROWS_PER_TILE
    srcs = (a_hbm, b_hbm, x2_hbm, mem_hbm)
    pltpu.sync_copy(ids_hbm.at[s], ids_v)

    def zrow(i, carry):
        zbuf[i, :] = jnp.zeros((16,), jnp.float32)
        return carry
    lax.fori_loop(0, ZROWS, zrow, 0)

    def flush_slice(slab, dcol0):
        return (acc.at[pl.ds(row0, ROWS_PER_TILE), :],
                table_hbm.at[slab, pl.ds(row0, ROWS_PER_TILE),
                             pl.ds(dcol0, 16)])

    def do_pass(src, scol0, dcol0, slab, first):
        def xsrc(j):
            return src.at[pl.ds(row0 + j * CH, CH), pl.ds(scol0, 16)]

        for b in range(NBUF):
            pltpu.async_copy(xsrc(b), buf.at[b], gsem.at[b])

        if not first:
            fs, fd = flush_slice(slab, dcol0)
            pltpu.make_async_copy(fs, fd, fsem).wait()

        for i in range(ROWS_PER_TILE // ZROWS):
            pltpu.async_copy(zbuf, acc.at[pl.ds(row0 + i * ZROWS, ZROWS), :],
                             zsem)
        for i in range(ROWS_PER_TILE // ZROWS):
            pltpu.make_async_copy(
                zbuf, acc.at[pl.ds(row0 + i * ZROWS, ZROWS), :], zsem).wait()
        plsc.subcore_barrier()

        def group(g, carry):
            for b in range(NBUF):
                j = g * NBUF + b
                pltpu.make_async_copy(xsrc(j), buf.at[b], gsem.at[b]).wait()
                pltpu.async_copy(buf.at[b], acc.at[ids_v.at[j]], ssem.at[b],
                                 add=True)
            for b in range(NBUF):
                j = g * NBUF + b
                pltpu.make_async_copy(buf.at[b], acc.at[ids_v.at[j]],
                                      ssem.at[b]).wait()
                jn = j + NBUF

                @pl.when(jn < NCHUNK_S)
                def _():
                    pltpu.async_copy(xsrc(jn), buf.at[b], gsem.at[b])
            return carry
        lax.fori_loop(0, NGROUP, group, 0)
        plsc.subcore_barrier()

        fs, fd = flush_slice(slab, dcol0)
        pltpu.async_copy(fs, fd, fsem)

    for p in range(NPASS):
        sa, ca = _SRC_OF_SLICE[2 * p]
        do_pass(srcs[sa], ca + c * 16, ((2 * p) % 8 + c) * 16,
                p // 4, p == 0)

    fs, fd = flush_slice(3, (14 % 8 + c) * 16)
    pltpu.make_async_copy(fs, fd, fsem).wait()


@functools.partial(
    pl.kernel,
    out_type=jax.ShapeDtypeStruct((N_G, 128), jnp.float32),
    mesh=_sc_mesh,
    compiler_params=pltpu.CompilerParams(use_tc_tiling_on_sc=True),
    scratch_types=[
        pltpu.VMEM((NCHUNK_G, CH_G), jnp.int32),
        pltpu.VMEM((CH_G, 128), jnp.float32),
        pltpu.SemaphoreType.DMA,
    ],
)
def _gather_kernel(upd_hbm, ids_hbm, out_hbm, ids_v, rows_v, sem):
    c = lax.axis_index("c")
    s = lax.axis_index("s")
    w = s * NCORE + c
    chunk0 = w * NCHUNK_G
    pltpu.sync_copy(ids_hbm.at[w], ids_v)

    def chunk(j, carry):
        pltpu.async_copy(upd_hbm.at[ids_v.at[j]], rows_v, sem).wait()
        pltpu.sync_copy(rows_v, out_hbm.at[pl.ds((chunk0 + j) * CH_G, CH_G), :])
        return carry
    lax.fori_loop(0, NCHUNK_G, chunk, 0)


def kernel(all_ids, mem_input, ts, mem_ts, mem, h, num_dst_nodes,
           time_w, time_b, w_ih, w_hh, b_ih, b_hh, proj_w, proj_b):
    ids_i32 = all_ids.astype(jnp.int32)
    ids_s = ids_i32.reshape(NSUB, NCHUNK_S, CH)
    ids_g = jnp.pad(ids_i32, (0, N_G - N)).reshape(NSUB * NCORE, NCHUNK_G, CH_G)

    ph = pl.pallas_call(
        _proj_body,
        grid=(N // BR,),
        in_specs=[
            pl.BlockSpec((BR, 256), lambda i: (i, 0)),
            pl.BlockSpec((256, 128), lambda i: (0, 0)),
            pl.BlockSpec((1, 128), lambda i: (0, 0)),
        ],
        out_specs=pl.BlockSpec((BR, 128), lambda i: (i, 0)),
        out_shape=jax.ShapeDtypeStruct((N, 128), jnp.float32),
    )(h, proj_w.T.astype(jnp.bfloat16), proj_b.reshape(1, 128))

    slab_spec = pl.BlockSpec((BR, 128), lambda i: (i, 0))
    slab_shape = jax.ShapeDtypeStruct((N, 128), jnp.float32)
    mi_a, mi_b, x2, mem_c = pl.pallas_call(
        _build_body,
        grid=(N // BR,),
        in_specs=[
            pl.BlockSpec((BR, 272), lambda i: (i, 0)),
            pl.BlockSpec((BR, 1), lambda i: (i, 0)),
            pl.BlockSpec((BR, 1), lambda i: (i, 0)),
            pl.BlockSpec((1, 100), lambda i: (0, 0)),
            pl.BlockSpec((1, 100), lambda i: (0, 0)),
            pl.BlockSpec((BR, 128), lambda i: (i, 0)),
        ],
        out_specs=[slab_spec, slab_spec, slab_spec, slab_spec],
        out_shape=[slab_shape, slab_shape, slab_shape, slab_shape],
    )(mem_input, ts.reshape(N, 1), mem_ts.reshape(N, 1),
      time_w.reshape(1, 100), time_b.reshape(1, 100), mem)

    table = _scatter_kernel(mi_a, mi_b, x2, mem_c, ids_s)

    wih_pad = jnp.zeros((384, 384), jnp.float32).at[:372, :].set(w_ih.T)
    upd = pl.pallas_call(
        _gru_body,
        grid=(N // BR,),
        in_specs=[
            pl.BlockSpec((4, BR, 128), lambda i: (0, i, 0)),
            pl.BlockSpec((384, 384), lambda i: (0, 0)),
            pl.BlockSpec((128, 384), lambda i: (0, 0)),
            pl.BlockSpec((1, 384), lambda i: (0, 0)),
            pl.BlockSpec((1, 384), lambda i: (0, 0)),
        ],
        out_specs=pl.BlockSpec((BR, 128), lambda i: (i, 0)),
        out_shape=jax.ShapeDtypeStruct((N, 128), jnp.float32),
    )(table, wih_pad.astype(jnp.bfloat16), w_hh.T.astype(jnp.bfloat16),
      b_ih.reshape(1, 384), b_hh.reshape(1, 384))

    restored = _gather_kernel(upd, ids_g)

    h_out = pl.pallas_call(
        _add_body,
        grid=(N // BR,),
        in_specs=[
            pl.BlockSpec((BR, 128), lambda i: (i, 0)),
            pl.BlockSpec((BR, 128), lambda i: (i, 0)),
        ],
        out_specs=pl.BlockSpec((BR, 128), lambda i: (i, 0)),
        out_shape=jax.ShapeDtypeStruct((N, 128), jnp.float32),
    )(restored, ph)

    nd = 50000
    last_updated_nid = all_ids[:nd] + (num_dst_nodes - nd)
    return last_updated_nid, restored[:nd], ts[:nd], h_out


# speedup vs baseline: 1.2609x; 1.0038x over previous
import functools

import jax
import jax.numpy as jnp
from jax import lax
from jax.experimental import pallas as pl
from jax.experimental.pallas import tpu as pltpu
from jax.experimental.pallas import tpu_sc as plsc

N = 100000
XW = 512
X2W = 128
CNT_COL = 383
NPASS = 16
CH = 125
NSUB = 16
NCORE = 2
ROWS_PER_TILE = N // NSUB
NCHUNK_S = ROWS_PER_TILE // CH
N_G = 100352
CH_G = 112
ROWS_PER_W = N_G // (NSUB * NCORE)
NCHUNK_G = ROWS_PER_W // CH_G
ZROWS = 250
NBUF = 10
NGROUP = NCHUNK_S // NBUF
BR = 1000

_SRC_OF_SLICE = [(s // 8, 16 * (s % 8)) for s in range(32)]


def _build_body(mi_ref, ts_ref, mts_ref, tw_ref, tb_ref, mem_ref,
                a_ref, b_ref, x_ref, m_ref):
    mi = mi_ref[...]
    a_ref[...] = mi[:, :128]
    b_ref[...] = mi[:, 128:256]
    dt = ts_ref[...] - mts_ref[...]
    tf = jnp.cos(dt * tw_ref[...] + tb_ref[...])
    br = tf.shape[0]
    x_ref[...] = jnp.concatenate([
        mi[:, 256:272],
        tf,
        jnp.zeros((br, 11), jnp.float32),
        jnp.ones((br, 1), jnp.float32),
    ], axis=1)
    m_ref[...] = mem_ref[...]


def _gru_body(t0_ref, t1_ref, t2_ref, t3_ref,
              wih_ref, whh_ref, bih_ref, bhh_ref, upd_ref):
    t2 = t2_ref[...]
    xa = jnp.concatenate([t0_ref[...], t1_ref[...], t2], axis=1)
    ma = t3_ref[...]
    cnt = t2[:, 127:128]
    hprev = ma / jnp.maximum(cnt, 1.0)
    gi = jnp.dot(xa.astype(jnp.bfloat16), wih_ref[...],
                 preferred_element_type=jnp.float32) + bih_ref[...]
    gh = jnp.dot(hprev.astype(jnp.bfloat16), whh_ref[...],
                 preferred_element_type=jnp.float32) + bhh_ref[...]
    r = jax.nn.sigmoid(gi[:, :128] + gh[:, :128])
    z = jax.nn.sigmoid(gi[:, 128:256] + gh[:, 128:256])
    n = jnp.tanh(gi[:, 256:] + r * gh[:, 256:])
    upd_ref[...] = (1.0 - z) * n + z * hprev


def _proj_body(h_ref, pw_ref, pb_ref, o_ref):
    o_ref[...] = (jnp.dot(h_ref[...].astype(jnp.bfloat16), pw_ref[...],
                          preferred_element_type=jnp.float32) + pb_ref[...])


def _add_body(r_ref, p_ref, o_ref):
    o_ref[...] = r_ref[...] + p_ref[...]


_sc_mesh = plsc.VectorSubcoreMesh(core_axis_name="c", subcore_axis_name="s")
_sc_params = pltpu.CompilerParams(use_tc_tiling_on_sc=False)


@functools.partial(
    pl.kernel,
    out_type=[jax.ShapeDtypeStruct((N, 128), jnp.float32)] * 4,
    mesh=_sc_mesh,
    compiler_params=_sc_params,
    scratch_types=[
        pltpu.VMEM_SHARED((N, 16), jnp.float32),
        pltpu.VMEM((NCHUNK_S, CH), jnp.int32),
        pltpu.VMEM((NBUF, CH, 16), jnp.float32),
        pltpu.VMEM((ZROWS, 16), jnp.float32),
        pltpu.SemaphoreType.DMA((NBUF,)),
        pltpu.SemaphoreType.DMA((NBUF,)),
        pltpu.SemaphoreType.DMA,
        pltpu.SemaphoreType.DMA,
    ],
)
def _scatter_kernel(a_hbm, b_hbm, x2_hbm, mem_hbm, ids_hbm,
                    t0_hbm, t1_hbm, t2_hbm, t3_hbm,
                    acc, ids_v, buf, zbuf, gsem, ssem, zsem, fsem):
    c = lax.axis_index("c")
    s = lax.axis_index("s")
    row0 = s * ROWS_PER_TILE
    srcs = (a_hbm, b_hbm, x2_hbm, mem_hbm)
    pltpu.sync_copy(ids_hbm.at[s], ids_v)

    def zrow(i, carry):
        zbuf[i, :] = jnp.zeros((16,), jnp.float32)
        return carry
    lax.fori_loop(0, ZROWS, zrow, 0)

    tbs = (t0_hbm, t1_hbm, t2_hbm, t3_hbm)

    def flush_slice(slab, dcol0):
        return (acc.at[pl.ds(row0, ROWS_PER_TILE), :],
                tbs[slab].at[pl.ds(row0, ROWS_PER_TILE), pl.ds(dcol0, 16)])

    def do_pass(src, scol0, dcol0, slab, first):
        def xsrc(j):
            return src.at[pl.ds(row0 + j * CH, CH), pl.ds(scol0, 16)]

        for b in range(NBUF):
            pltpu.async_copy(xsrc(b), buf.at[b], gsem.at[b])

        if not first:
            fs, fd = flush_slice(slab, dcol0)
            pltpu.make_async_copy(fs, fd, fsem).wait()

        for i in range(ROWS_PER_TILE // ZROWS):
            pltpu.async_copy(zbuf, acc.at[pl.ds(row0 + i * ZROWS, ZROWS), :],
                             zsem)
        for i in range(ROWS_PER_TILE // ZROWS):
            pltpu.make_async_copy(
                zbuf, acc.at[pl.ds(row0 + i * ZROWS, ZROWS), :], zsem).wait()
        plsc.subcore_barrier()

        def group(g, carry):
            for b in range(NBUF):
                j = g * NBUF + b
                pltpu.make_async_copy(xsrc(j), buf.at[b], gsem.at[b]).wait()
                pltpu.async_copy(buf.at[b], acc.at[ids_v.at[j]], ssem.at[b],
                                 add=True)
            for b in range(NBUF):
                j = g * NBUF + b
                pltpu.make_async_copy(buf.at[b], acc.at[ids_v.at[j]],
                                      ssem.at[b]).wait()
                jn = j + NBUF

                @pl.when(jn < NCHUNK_S)
                def _():
                    pltpu.async_copy(xsrc(jn), buf.at[b], gsem.at[b])
            return carry
        lax.fori_loop(0, NGROUP, group, 0)
        plsc.subcore_barrier()

        fs, fd = flush_slice(slab, dcol0)
        pltpu.async_copy(fs, fd, fsem)

    for p in range(NPASS):
        sa, ca = _SRC_OF_SLICE[2 * p]
        do_pass(srcs[sa], ca + c * 16, ((2 * p) % 8 + c) * 16,
                p // 4, p == 0)

    fs, fd = flush_slice(3, (14 % 8 + c) * 16)
    pltpu.make_async_copy(fs, fd, fsem).wait()


@functools.partial(
    pl.kernel,
    out_type=jax.ShapeDtypeStruct((N_G, 128), jnp.float32),
    mesh=_sc_mesh,
    compiler_params=pltpu.CompilerParams(use_tc_tiling_on_sc=True),
    scratch_types=[
        pltpu.VMEM((NCHUNK_G, CH_G), jnp.int32),
        pltpu.VMEM((CH_G, 128), jnp.float32),
        pltpu.SemaphoreType.DMA,
    ],
)
def _gather_kernel(upd_hbm, ids_hbm, out_hbm, ids_v, rows_v, sem):
    c = lax.axis_index("c")
    s = lax.axis_index("s")
    w = s * NCORE + c
    chunk0 = w * NCHUNK_G
    pltpu.sync_copy(ids_hbm.at[w], ids_v)

    def chunk(j, carry):
        pltpu.async_copy(upd_hbm.at[ids_v.at[j]], rows_v, sem).wait()
        pltpu.sync_copy(rows_v, out_hbm.at[pl.ds((chunk0 + j) * CH_G, CH_G), :])
        return carry
    lax.fori_loop(0, NCHUNK_G, chunk, 0)


def kernel(all_ids, mem_input, ts, mem_ts, mem, h, num_dst_nodes,
           time_w, time_b, w_ih, w_hh, b_ih, b_hh, proj_w, proj_b):
    ids_i32 = all_ids.astype(jnp.int32)
    ids_s = ids_i32.reshape(NSUB, NCHUNK_S, CH)
    ids_g = jnp.pad(ids_i32, (0, N_G - N)).reshape(NSUB * NCORE, NCHUNK_G, CH_G)

    ph = pl.pallas_call(
        _proj_body,
        grid=(N // BR,),
        in_specs=[
            pl.BlockSpec((BR, 256), lambda i: (i, 0)),
            pl.BlockSpec((256, 128), lambda i: (0, 0)),
            pl.BlockSpec((1, 128), lambda i: (0, 0)),
        ],
        out_specs=pl.BlockSpec((BR, 128), lambda i: (i, 0)),
        out_shape=jax.ShapeDtypeStruct((N, 128), jnp.float32),
    )(h, proj_w.T.astype(jnp.bfloat16), proj_b.reshape(1, 128))

    slab_spec = pl.BlockSpec((BR, 128), lambda i: (i, 0))
    slab_shape = jax.ShapeDtypeStruct((N, 128), jnp.float32)
    mi_a, mi_b, x2, mem_c = pl.pallas_call(
        _build_body,
        grid=(N // BR,),
        in_specs=[
            pl.BlockSpec((BR, 272), lambda i: (i, 0)),
            pl.BlockSpec((BR, 1), lambda i: (i, 0)),
            pl.BlockSpec((BR, 1), lambda i: (i, 0)),
            pl.BlockSpec((1, 100), lambda i: (0, 0)),
            pl.BlockSpec((1, 100), lambda i: (0, 0)),
            pl.BlockSpec((BR, 128), lambda i: (i, 0)),
        ],
        out_specs=[slab_spec, slab_spec, slab_spec, slab_spec],
        out_shape=[slab_shape, slab_shape, slab_shape, slab_shape],
    )(mem_input, ts.reshape(N, 1), mem_ts.reshape(N, 1),
      time_w.reshape(1, 100), time_b.reshape(1, 100), mem)

    t0, t1, t2, t3 = _scatter_kernel(mi_a, mi_b, x2, mem_c, ids_s)

    wih_pad = jnp.zeros((384, 384), jnp.float32).at[:372, :].set(w_ih.T)
    upd = pl.pallas_call(
        _gru_body,
        grid=(N // BR,),
        in_specs=[
            slab_spec, slab_spec, slab_spec, slab_spec,
            pl.BlockSpec((384, 384), lambda i: (0, 0)),
            pl.BlockSpec((128, 384), lambda i: (0, 0)),
            pl.BlockSpec((1, 384), lambda i: (0, 0)),
            pl.BlockSpec((1, 384), lambda i: (0, 0)),
        ],
        out_specs=pl.BlockSpec((BR, 128), lambda i: (i, 0)),
        out_shape=jax.ShapeDtypeStruct((N, 128), jnp.float32),
    )(t0, t1, t2, t3, wih_pad.astype(jnp.bfloat16), w_hh.T.astype(jnp.bfloat16),
      b_ih.reshape(1, 384), b_hh.reshape(1, 384))

    restored = _gather_kernel(upd, ids_g)

    h_out = pl.pallas_call(
        _add_body,
        grid=(N // BR,),
        in_specs=[
            pl.BlockSpec((BR, 128), lambda i: (i, 0)),
            pl.BlockSpec((BR, 128), lambda i: (i, 0)),
        ],
        out_specs=pl.BlockSpec((BR, 128), lambda i: (i, 0)),
        out_shape=jax.ShapeDtypeStruct((N, 128), jnp.float32),
    )(restored, ph)

    nd = 50000
    last_updated_nid = all_ids[:nd] + (num_dst_nodes - nd)
    return last_updated_nid, restored[:nd], ts[:nd], h_out


# speedup vs baseline: 1.2986x; 1.0299x over previous
import functools

import jax
import jax.numpy as jnp
from jax import lax
from jax.experimental import pallas as pl
from jax.experimental.pallas import tpu as pltpu
from jax.experimental.pallas import tpu_sc as plsc

N = 100000
XW = 512
X2W = 128
CNT_COL = 383
NPASS = 16
CH = 125
NSUB = 16
NCORE = 2
ROWS_PER_TILE = N // NSUB
NCHUNK_S = ROWS_PER_TILE // CH
N_G = 100352
CH_G = 112
ROWS_PER_W = N_G // (NSUB * NCORE)
NCHUNK_G = ROWS_PER_W // CH_G
ZROWS = 250
NBUF = 10
NGROUP = NCHUNK_S // NBUF
BR = 1000
BR2 = 2000

_SRC_OF_SLICE = [(s // 8, 16 * (s % 8)) for s in range(32)]


def _build_body(mi_ref, ts_ref, mts_ref, tw_ref, tb_ref, mem_ref,
                a_ref, b_ref, x_ref, m_ref):
    mi = mi_ref[...]
    a_ref[...] = mi[:, :128]
    b_ref[...] = mi[:, 128:256]
    dt = ts_ref[...] - mts_ref[...]
    tf = jnp.cos(dt * tw_ref[...] + tb_ref[...])
    br = tf.shape[0]
    x_ref[...] = jnp.concatenate([
        mi[:, 256:272],
        tf,
        jnp.zeros((br, 11), jnp.float32),
        jnp.ones((br, 1), jnp.float32),
    ], axis=1)
    m_ref[...] = mem_ref[...]


def _gru_body(t0_ref, t1_ref, t2_ref, t3_ref,
              wih_ref, whh_ref, bih_ref, bhh_ref, upd_ref):
    t2 = t2_ref[...]
    xa = jnp.concatenate([t0_ref[...], t1_ref[...], t2], axis=1)
    ma = t3_ref[...]
    cnt = t2[:, 127:128]
    hprev = ma / jnp.maximum(cnt, 1.0)
    gi = jnp.dot(xa.astype(jnp.bfloat16), wih_ref[...],
                 preferred_element_type=jnp.float32) + bih_ref[...]
    gh = jnp.dot(hprev.astype(jnp.bfloat16), whh_ref[...],
                 preferred_element_type=jnp.float32) + bhh_ref[...]
    r = jax.nn.sigmoid(gi[:, :128] + gh[:, :128])
    z = jax.nn.sigmoid(gi[:, 128:256] + gh[:, 128:256])
    n = jnp.tanh(gi[:, 256:] + r * gh[:, 256:])
    upd_ref[...] = (1.0 - z) * n + z * hprev


def _proj_body(h_ref, pw_ref, pb_ref, o_ref):
    o_ref[...] = (jnp.dot(h_ref[...].astype(jnp.bfloat16), pw_ref[...],
                          preferred_element_type=jnp.float32) + pb_ref[...])


def _add_body(r_ref, p_ref, o_ref):
    o_ref[...] = r_ref[...] + p_ref[...]


_sc_mesh = plsc.VectorSubcoreMesh(core_axis_name="c", subcore_axis_name="s")
_sc_params = pltpu.CompilerParams(use_tc_tiling_on_sc=False)


@functools.partial(
    pl.kernel,
    out_type=[jax.ShapeDtypeStruct((N, 128), jnp.float32)] * 4,
    mesh=_sc_mesh,
    compiler_params=_sc_params,
    scratch_types=[
        pltpu.VMEM_SHARED((N, 16), jnp.float32),
        pltpu.VMEM((NCHUNK_S, CH), jnp.int32),
        pltpu.VMEM((NBUF, CH, 16), jnp.float32),
        pltpu.VMEM((ZROWS, 16), jnp.float32),
        pltpu.SemaphoreType.DMA((NBUF,)),
        pltpu.SemaphoreType.DMA((NBUF,)),
        pltpu.SemaphoreType.DMA,
        pltpu.SemaphoreType.DMA,
    ],
)
def _scatter_kernel(a_hbm, b_hbm, x2_hbm, mem_hbm, ids_hbm,
                    t0_hbm, t1_hbm, t2_hbm, t3_hbm,
                    acc, ids_v, buf, zbuf, gsem, ssem, zsem, fsem):
    c = lax.axis_index("c")
    s = lax.axis_index("s")
    row0 = s * ROWS_PER_TILE
    srcs = (a_hbm, b_hbm, x2_hbm, mem_hbm)
    pltpu.sync_copy(ids_hbm.at[s], ids_v)

    def zrow(i, carry):
        zbuf[i, :] = jnp.zeros((16,), jnp.float32)
        return carry
    lax.fori_loop(0, ZROWS, zrow, 0)

    tbs = (t0_hbm, t1_hbm, t2_hbm, t3_hbm)

    def flush_slice(slab, dcol0):
        return (acc.at[pl.ds(row0, ROWS_PER_TILE), :],
                tbs[slab].at[pl.ds(row0, ROWS_PER_TILE), pl.ds(dcol0, 16)])

    def do_pass(src, scol0, dcol0, slab, first):
        def xsrc(j):
            return src.at[pl.ds(row0 + j * CH, CH), pl.ds(scol0, 16)]

        for b in range(NBUF):
            pltpu.async_copy(xsrc(b), buf.at[b], gsem.at[b])

        if not first:
            fs, fd = flush_slice(slab, dcol0)
            pltpu.make_async_copy(fs, fd, fsem).wait()

        for i in range(ROWS_PER_TILE // ZROWS):
            pltpu.async_copy(zbuf, acc.at[pl.ds(row0 + i * ZROWS, ZROWS), :],
                             zsem)
        for i in range(ROWS_PER_TILE // ZROWS):
            pltpu.make_async_copy(
                zbuf, acc.at[pl.ds(row0 + i * ZROWS, ZROWS), :], zsem).wait()
        plsc.subcore_barrier()

        def group(g, carry):
            for b in range(NBUF):
                j = g * NBUF + b
                pltpu.make_async_copy(xsrc(j), buf.at[b], gsem.at[b]).wait()
                pltpu.async_copy(buf.at[b], acc.at[ids_v.at[j]], ssem.at[b],
                                 add=True)
            for b in range(NBUF):
                j = g * NBUF + b
                pltpu.make_async_copy(buf.at[b], acc.at[ids_v.at[j]],
                                      ssem.at[b]).wait()
                jn = j + NBUF

                @pl.when(jn < NCHUNK_S)
                def _():
                    pltpu.async_copy(xsrc(jn), buf.at[b], gsem.at[b])
            return carry
        lax.fori_loop(0, NGROUP, group, 0)
        plsc.subcore_barrier()

        fs, fd = flush_slice(slab, dcol0)
        pltpu.async_copy(fs, fd, fsem)

    for p in range(NPASS):
        sa, ca = _SRC_OF_SLICE[2 * p]
        do_pass(srcs[sa], ca + c * 16, ((2 * p) % 8 + c) * 16,
                p // 4, p == 0)

    fs, fd = flush_slice(3, (14 % 8 + c) * 16)
    pltpu.make_async_copy(fs, fd, fsem).wait()


@functools.partial(
    pl.kernel,
    out_type=jax.ShapeDtypeStruct((N_G, 128), jnp.float32),
    mesh=_sc_mesh,
    compiler_params=pltpu.CompilerParams(use_tc_tiling_on_sc=True),
    scratch_types=[
        pltpu.VMEM((NCHUNK_G, CH_G), jnp.int32),
        pltpu.VMEM((CH_G, 128), jnp.float32),
        pltpu.SemaphoreType.DMA,
    ],
)
def _gather_kernel(upd_hbm, ids_hbm, out_hbm, ids_v, rows_v, sem):
    c = lax.axis_index("c")
    s = lax.axis_index("s")
    w = s * NCORE + c
    chunk0 = w * NCHUNK_G
    pltpu.sync_copy(ids_hbm.at[w], ids_v)

    def chunk(j, carry):
        pltpu.async_copy(upd_hbm.at[ids_v.at[j]], rows_v, sem).wait()
        pltpu.sync_copy(rows_v, out_hbm.at[pl.ds((chunk0 + j) * CH_G, CH_G), :])
        return carry
    lax.fori_loop(0, NCHUNK_G, chunk, 0)


def kernel(all_ids, mem_input, ts, mem_ts, mem, h, num_dst_nodes,
           time_w, time_b, w_ih, w_hh, b_ih, b_hh, proj_w, proj_b):
    ids_i32 = all_ids.astype(jnp.int32)
    ids_s = ids_i32.reshape(NSUB, NCHUNK_S, CH)
    ids_g = jnp.pad(ids_i32, (0, N_G - N)).reshape(NSUB * NCORE, NCHUNK_G, CH_G)

    ph = pl.pallas_call(
        _proj_body,
        grid=(N // BR2,),
        in_specs=[
            pl.BlockSpec((BR2, 256), lambda i: (i, 0)),
            pl.BlockSpec((256, 128), lambda i: (0, 0)),
            pl.BlockSpec((1, 128), lambda i: (0, 0)),
        ],
        out_specs=pl.BlockSpec((BR2, 128), lambda i: (i, 0)),
        out_shape=jax.ShapeDtypeStruct((N, 128), jnp.float32),
    )(h, proj_w.T.astype(jnp.bfloat16), proj_b.reshape(1, 128))

    slab_spec = pl.BlockSpec((BR, 128), lambda i: (i, 0))
    slab_shape = jax.ShapeDtypeStruct((N, 128), jnp.float32)
    mi_a, mi_b, x2, mem_c = pl.pallas_call(
        _build_body,
        grid=(N // BR,),
        in_specs=[
            pl.BlockSpec((BR, 272), lambda i: (i, 0)),
            pl.BlockSpec((BR, 1), lambda i: (i, 0)),
            pl.BlockSpec((BR, 1), lambda i: (i, 0)),
            pl.BlockSpec((1, 100), lambda i: (0, 0)),
            pl.BlockSpec((1, 100), lambda i: (0, 0)),
            pl.BlockSpec((BR, 128), lambda i: (i, 0)),
        ],
        out_specs=[slab_spec, slab_spec, slab_spec, slab_spec],
        out_shape=[slab_shape, slab_shape, slab_shape, slab_shape],
    )(mem_input, ts.reshape(N, 1), mem_ts.reshape(N, 1),
      time_w.reshape(1, 100), time_b.reshape(1, 100), mem)

    t0, t1, t2, t3 = _scatter_kernel(mi_a, mi_b, x2, mem_c, ids_s)

    wih_pad = jnp.zeros((384, 384), jnp.float32).at[:372, :].set(w_ih.T)
    upd = pl.pallas_call(
        _gru_body,
        grid=(N // BR,),
        in_specs=[
            slab_spec, slab_spec, slab_spec, slab_spec,
            pl.BlockSpec((384, 384), lambda i: (0, 0)),
            pl.BlockSpec((128, 384), lambda i: (0, 0)),
            pl.BlockSpec((1, 384), lambda i: (0, 0)),
            pl.BlockSpec((1, 384), lambda i: (0, 0)),
        ],
        out_specs=pl.BlockSpec((BR, 128), lambda i: (i, 0)),
        out_shape=jax.ShapeDtypeStruct((N, 128), jnp.float32),
    )(t0, t1, t2, t3, wih_pad.astype(jnp.bfloat16), w_hh.T.astype(jnp.bfloat16),
      b_ih.reshape(1, 384), b_hh.reshape(1, 384))

    restored = _gather_kernel(upd, ids_g)

    h_out = pl.pallas_call(
        _add_body,
        grid=(N // BR2,),
        in_specs=[
            pl.BlockSpec((BR2, 128), lambda i: (i, 0)),
            pl.BlockSpec((BR2, 128), lambda i: (i, 0)),
        ],
        out_specs=pl.BlockSpec((BR2, 128), lambda i: (i, 0)),
        out_shape=jax.ShapeDtypeStruct((N, 128), jnp.float32),
    )(restored, ph)

    nd = 50000
    last_updated_nid = all_ids[:nd] + (num_dst_nodes - nd)
    return last_updated_nid, restored[:nd], ts[:nd], h_out
